# baseline (speedup 1.0000x reference)
import functools

import jax
import jax.numpy as jnp
from jax import lax
from jax.experimental import pallas as pl
from jax.experimental.pallas import tpu as pltpu
from jax.experimental.pallas import tpu_sc as plsc

N_FACTORS = 64
N_ROWS = 1000001
BATCH = 16384
NC, NS = 2, 16
NW = NC * NS
BPW = BATCH // NW

_sc_mesh = plsc.VectorSubcoreMesh(core_axis_name="c", subcore_axis_name="s")


@functools.partial(
    pl.kernel,
    mesh=_sc_mesh,
    compiler_params=pltpu.CompilerParams(use_tc_tiling_on_sc=True),
    out_type=(
        jax.ShapeDtypeStruct((BATCH, 128), jnp.float32),
        jax.ShapeDtypeStruct((BATCH, 128), jnp.float32),
    ),
    scratch_types=[
        pltpu.VMEM((BPW,), jnp.int32),
        pltpu.VMEM((BPW,), jnp.int32),
        pltpu.VMEM((BPW, 128), jnp.float32),
        pltpu.SemaphoreType.DMA,
    ],
)
def _sc_gather(users_hbm, items_hbm, utab_hbm, itab_hbm, u_out, v_out,
               uidx_v, iidx_v, rows_v, sem):
    wid = lax.axis_index("s") * NC + lax.axis_index("c")
    base = wid * BPW
    pltpu.sync_copy(users_hbm.at[pl.ds(base, BPW)], uidx_v)
    pltpu.sync_copy(items_hbm.at[pl.ds(base, BPW)], iidx_v)

    def gather_one(tab, idx_v, out):
        def body(g, _):
            vec = idx_v[pl.ds(g * 16, 16)]
            for l in range(16):
                r = vec[l]
                pltpu.make_async_copy(
                    tab.at[r], rows_v.at[g * 16 + l, pl.ds(0, N_FACTORS)],
                    sem).start()
            return 0
        lax.fori_loop(0, BPW // 16, body, 0)
        pltpu.make_async_copy(
            out.at[pl.ds(0, BPW // 2)],
            rows_v.at[pl.ds(0, BPW // 2)], sem).wait()
        pltpu.sync_copy(rows_v, out.at[pl.ds(base, BPW)])

    gather_one(utab_hbm, uidx_v, u_out)
    gather_one(itab_hbm, iidx_v, v_out)


TBLK = 2048


def _transpose_body(inT_ref, out_ref):
    out_ref[...] = inT_ref[...].T


def _transpose_table(tabT):
    grid = (pl.cdiv(N_ROWS, TBLK),)
    return pl.pallas_call(
        _transpose_body,
        grid=grid,
        in_specs=[pl.BlockSpec((N_FACTORS, TBLK), lambda i: (0, i))],
        out_specs=pl.BlockSpec((TBLK, N_FACTORS), lambda i: (i, 0)),
        out_shape=jax.ShapeDtypeStruct((N_ROWS, N_FACTORS), jnp.float32),
    )(tabT)


BLK = 2048


def _mlp_body(u_ref, v_ref, w1a_ref, w1b_ref, b1_ref, w2_ref, b2_ref,
              wf_ref, bf_ref, out_ref):
    u = u_ref[:, :N_FACTORS]
    v = v_ref[:, :N_FACTORS]
    h = jnp.dot(u, w1a_ref[...], preferred_element_type=jnp.float32)
    h += jnp.dot(v, w1b_ref[...], preferred_element_type=jnp.float32)
    h = jnp.maximum(h + b1_ref[...], 0.0)
    h = jnp.dot(h, w2_ref[...], preferred_element_type=jnp.float32)
    h = jnp.maximum(h + b2_ref[...], 0.0)
    out_ref[...] = jnp.dot(h, wf_ref[...],
                           preferred_element_type=jnp.float32) + bf_ref[...]


def _mlp(u, v, W1, b1, W2, b2, Wf, bf):
    w1a, w1b = W1[:N_FACTORS], W1[N_FACTORS:]
    grid = (BATCH // BLK,)
    fixed = lambda i: (0, 0)
    return pl.pallas_call(
        _mlp_body,
        grid=grid,
        in_specs=[
            pl.BlockSpec((BLK, 128), lambda i: (i, 0)),
            pl.BlockSpec((BLK, 128), lambda i: (i, 0)),
            pl.BlockSpec((N_FACTORS, 256), fixed),
            pl.BlockSpec((N_FACTORS, 256), fixed),
            pl.BlockSpec((1, 256), fixed),
            pl.BlockSpec((256, 128), fixed),
            pl.BlockSpec((1, 128), fixed),
            pl.BlockSpec((128, 1), fixed),
            pl.BlockSpec((1, 1), fixed),
        ],
        out_specs=pl.BlockSpec((BLK, 1), lambda i: (i, 0)),
        out_shape=jax.ShapeDtypeStruct((BATCH, 1), jnp.float32),
    )(u, v, w1a, w1b, b1.reshape(1, 256), W2, b2.reshape(1, 128),
      Wf, bf.reshape(1, 1))


def kernel(users, items, user_emb, item_emb, W1, b1, W2, b2, Wf, bf):
    utab = _transpose_table(user_emb.T)
    itab = _transpose_table(item_emb.T)
    u, v = _sc_gather(users.astype(jnp.int32), items.astype(jnp.int32),
                      utab, itab)
    out = _mlp(u, v, W1, b1, W2, b2, Wf, bf)
    return out.reshape(BATCH)


# speedup vs baseline: 1.4916x; 1.4916x over previous
import functools

import jax
import jax.numpy as jnp
from jax import lax
from jax.experimental import pallas as pl
from jax.experimental.pallas import tpu as pltpu
from jax.experimental.pallas import tpu_sc as plsc

N_FACTORS = 64
N_ROWS = 1000001
BATCH = 16384
TB = 2048
NTB = 245
HP = TB * NTB
NC, NS = 2, 16
NW = NC * NS
BPW = BATCH // NW

_sc_mesh = plsc.VectorSubcoreMesh(core_axis_name="c", subcore_axis_name="s")


@functools.partial(
    pl.kernel,
    mesh=_sc_mesh,
    compiler_params=pltpu.CompilerParams(use_tc_tiling_on_sc=True),
    out_type=(
        jax.ShapeDtypeStruct((BATCH, 128), jnp.float32),
        jax.ShapeDtypeStruct((BATCH, 128), jnp.float32),
    ),
    scratch_types=[
        pltpu.VMEM((BPW,), jnp.int32),
        pltpu.VMEM((BPW,), jnp.int32),
        pltpu.VMEM((BPW, 128), jnp.float32),
        pltpu.SemaphoreType.DMA,
    ],
)
def _sc_gather(users_hbm, items_hbm, utab_hbm, itab_hbm, u_out, v_out,
               uidx_v, iidx_v, rows_v, sem):
    wid = lax.axis_index("s") * NC + lax.axis_index("c")
    base = wid * BPW
    pltpu.sync_copy(users_hbm.at[pl.ds(base, BPW)], uidx_v)
    pltpu.sync_copy(items_hbm.at[pl.ds(base, BPW)], iidx_v)

    def gather_one(tab, idx_v, out):
        def body(g, _):
            vec = idx_v[pl.ds(g * 16, 16)]
            vec = jnp.where(vec >= HP, vec - HP, vec)
            for l in range(16):
                p = vec[l]
                pltpu.make_async_copy(tab.at[p], rows_v.at[g * 16 + l],
                                      sem).start()
            return 0
        lax.fori_loop(0, BPW // 16, body, 0)
        pltpu.make_async_copy(out.at[pl.ds(0, BPW)], rows_v, sem).wait()
        pltpu.sync_copy(rows_v, out.at[pl.ds(base, BPW)])

    gather_one(utab_hbm, uidx_v, u_out)
    gather_one(itab_hbm, iidx_v, v_out)


def _pair_body(a_ref, b_ref, eye_ref, out_ref):
    dn = (((0,), (0,)), ((), ()))
    a16 = a_ref[...].astype(jnp.bfloat16)
    b16 = b_ref[...].astype(jnp.bfloat16)
    e = eye_ref[...]
    out_ref[:, :N_FACTORS] = lax.dot_general(
        a16, e, dn, preferred_element_type=jnp.float32)
    out_ref[:, N_FACTORS:] = lax.dot_general(
        b16, e, dn, preferred_element_type=jnp.float32)


def _pair_table(tabT, eye16):
    return pl.pallas_call(
        _pair_body,
        grid=(NTB,),
        in_specs=[
            pl.BlockSpec((N_FACTORS, TB), lambda i: (0, i)),
            pl.BlockSpec((N_FACTORS, TB),
                         lambda i: (0, jnp.minimum(i + NTB, NTB * 2 - 2))),
            pl.BlockSpec((N_FACTORS, N_FACTORS), lambda i: (0, 0)),
        ],
        out_specs=pl.BlockSpec((TB, 128), lambda i: (i, 0)),
        out_shape=jax.ShapeDtypeStruct((HP, 128), jnp.float32),
    )(tabT, tabT, eye16)


BLK = 2048


def _mlp_body(u_ref, v_ref, mu_ref, mv_ref, w1a_ref, w1b_ref, b1_ref,
              w2_ref, b2_ref, wf_ref, bf_ref, out_ref):
    u = jnp.where(mu_ref[...] > 0.5, u_ref[:, N_FACTORS:],
                  u_ref[:, :N_FACTORS])
    v = jnp.where(mv_ref[...] > 0.5, v_ref[:, N_FACTORS:],
                  v_ref[:, :N_FACTORS])
    h = jnp.dot(u, w1a_ref[...], preferred_element_type=jnp.float32)
    h += jnp.dot(v, w1b_ref[...], preferred_element_type=jnp.float32)
    h = jnp.maximum(h + b1_ref[...], 0.0)
    h = jnp.dot(h, w2_ref[...], preferred_element_type=jnp.float32)
    h = jnp.maximum(h + b2_ref[...], 0.0)
    out_ref[...] = jnp.dot(h, wf_ref[...],
                           preferred_element_type=jnp.float32) + bf_ref[...]


def _mlp(u, v, mu, mv, W1, b1, W2, b2, Wf, bf):
    w1a, w1b = W1[:N_FACTORS], W1[N_FACTORS:]
    grid = (BATCH // BLK,)
    fixed = lambda i: (0, 0)
    return pl.pallas_call(
        _mlp_body,
        grid=grid,
        in_specs=[
            pl.BlockSpec((BLK, 128), lambda i: (i, 0)),
            pl.BlockSpec((BLK, 128), lambda i: (i, 0)),
            pl.BlockSpec((BLK, 1), lambda i: (i, 0)),
            pl.BlockSpec((BLK, 1), lambda i: (i, 0)),
            pl.BlockSpec((N_FACTORS, 256), fixed),
            pl.BlockSpec((N_FACTORS, 256), fixed),
            pl.BlockSpec((1, 256), fixed),
            pl.BlockSpec((256, 128), fixed),
            pl.BlockSpec((1, 128), fixed),
            pl.BlockSpec((128, 1), fixed),
            pl.BlockSpec((1, 1), fixed),
        ],
        out_specs=pl.BlockSpec((BLK, 1), lambda i: (i, 0)),
        out_shape=jax.ShapeDtypeStruct((BATCH, 1), jnp.float32),
    )(u, v, mu, mv, w1a, w1b, b1.reshape(1, 256), W2, b2.reshape(1, 128),
      Wf, bf.reshape(1, 1))


def kernel(users, items, user_emb, item_emb, W1, b1, W2, b2, Wf, bf):
    users = users.astype(jnp.int32)
    items = items.astype(jnp.int32)
    eye16 = jnp.eye(N_FACTORS, dtype=jnp.bfloat16)
    utab = _pair_table(user_emb.T, eye16)
    itab = _pair_table(item_emb.T, eye16)
    u, v = _sc_gather(users, items, utab, itab)
    mu = (users >= HP).astype(jnp.float32).reshape(BATCH, 1)
    mv = (items >= HP).astype(jnp.float32).reshape(BATCH, 1)
    out = _mlp(u, v, mu, mv, W1, b1, W2, b2, Wf, bf)
    return out.reshape(BATCH)


# speedup vs baseline: 1.6136x; 1.0818x over previous
import functools

import jax
import jax.numpy as jnp
from jax import lax
from jax.experimental import pallas as pl
from jax.experimental.pallas import tpu as pltpu
from jax.experimental.pallas import tpu_sc as plsc

N_FACTORS = 64
N_ROWS = 1000001
BATCH = 16384
TB = 2048
NTB = 245
HP = TB * NTB
NC, NS = 2, 16
NW = NC * NS
BPW = BATCH // NW

_sc_mesh = plsc.VectorSubcoreMesh(core_axis_name="c", subcore_axis_name="s")


@functools.partial(
    pl.kernel,
    mesh=_sc_mesh,
    compiler_params=pltpu.CompilerParams(use_tc_tiling_on_sc=True),
    out_type=(
        jax.ShapeDtypeStruct((BATCH, 128), jnp.float32),
        jax.ShapeDtypeStruct((BATCH, 128), jnp.float32),
    ),
    scratch_types=[
        pltpu.VMEM((BPW,), jnp.int32),
        pltpu.VMEM((BPW,), jnp.int32),
        pltpu.VMEM((BPW, 128), jnp.float32),
        pltpu.SemaphoreType.DMA,
    ],
)
def _sc_gather(users_hbm, items_hbm, utab_hbm, itab_hbm, u_out, v_out,
               uidx_v, iidx_v, rows_v, sem):
    wid = lax.axis_index("s") * NC + lax.axis_index("c")
    base = wid * BPW
    pltpu.sync_copy(users_hbm.at[pl.ds(base, BPW)], uidx_v)
    pltpu.sync_copy(items_hbm.at[pl.ds(base, BPW)], iidx_v)

    def gather_one(tab, idx_v, out):
        def body(g, _):
            vec = idx_v[pl.ds(g * 16, 16)]
            vec = jnp.where(vec >= HP, vec - HP, vec)
            for l in range(16):
                p = vec[l]
                pltpu.make_async_copy(tab.at[p], rows_v.at[g * 16 + l],
                                      sem).start()
            return 0
        lax.fori_loop(0, BPW // 16, body, 0)
        pltpu.make_async_copy(out.at[pl.ds(0, BPW)], rows_v, sem).wait()
        pltpu.sync_copy(rows_v, out.at[pl.ds(base, BPW)])

    gather_one(utab_hbm, uidx_v, u_out)
    gather_one(itab_hbm, iidx_v, v_out)


def _pair_body(a_ref, b_ref, eye_ref, out_ref):
    dn = (((0,), (0,)), ((), ()))
    c16 = jnp.concatenate(
        [a_ref[...], b_ref[...]], axis=0).astype(jnp.bfloat16)
    out_ref[...] = lax.dot_general(
        c16, eye_ref[...], dn, preferred_element_type=jnp.float32)


def _pair_table(tabT, eye16):
    return pl.pallas_call(
        _pair_body,
        grid=(NTB,),
        in_specs=[
            pl.BlockSpec((N_FACTORS, TB), lambda i: (0, i)),
            pl.BlockSpec((N_FACTORS, TB),
                         lambda i: (0, jnp.minimum(i + NTB, NTB * 2 - 2))),
            pl.BlockSpec((128, 128), lambda i: (0, 0)),
        ],
        out_specs=pl.BlockSpec((TB, 128), lambda i: (i, 0)),
        out_shape=jax.ShapeDtypeStruct((HP, 128), jnp.float32),
    )(tabT, tabT, eye16)


BLK = 2048


def _mlp_body(u_ref, v_ref, mu_ref, mv_ref, w1a_ref, w1b_ref, b1_ref,
              w2_ref, b2_ref, wf_ref, bf_ref, out_ref):
    u = jnp.where(mu_ref[...] > 0.5, u_ref[:, N_FACTORS:],
                  u_ref[:, :N_FACTORS])
    v = jnp.where(mv_ref[...] > 0.5, v_ref[:, N_FACTORS:],
                  v_ref[:, :N_FACTORS])
    h = jnp.dot(u, w1a_ref[...], preferred_element_type=jnp.float32)
    h += jnp.dot(v, w1b_ref[...], preferred_element_type=jnp.float32)
    h = jnp.maximum(h + b1_ref[...], 0.0)
    h = jnp.dot(h, w2_ref[...], preferred_element_type=jnp.float32)
    h = jnp.maximum(h + b2_ref[...], 0.0)
    out_ref[...] = jnp.dot(h, wf_ref[...],
                           preferred_element_type=jnp.float32) + bf_ref[...]


def _mlp(u, v, mu, mv, W1, b1, W2, b2, Wf, bf):
    w1a, w1b = W1[:N_FACTORS], W1[N_FACTORS:]
    grid = (BATCH // BLK,)
    fixed = lambda i: (0, 0)
    return pl.pallas_call(
        _mlp_body,
        grid=grid,
        in_specs=[
            pl.BlockSpec((BLK, 128), lambda i: (i, 0)),
            pl.BlockSpec((BLK, 128), lambda i: (i, 0)),
            pl.BlockSpec((BLK, 1), lambda i: (i, 0)),
            pl.BlockSpec((BLK, 1), lambda i: (i, 0)),
            pl.BlockSpec((N_FACTORS, 256), fixed),
            pl.BlockSpec((N_FACTORS, 256), fixed),
            pl.BlockSpec((1, 256), fixed),
            pl.BlockSpec((256, 128), fixed),
            pl.BlockSpec((1, 128), fixed),
            pl.BlockSpec((128, 1), fixed),
            pl.BlockSpec((1, 1), fixed),
        ],
        out_specs=pl.BlockSpec((BLK, 1), lambda i: (i, 0)),
        out_shape=jax.ShapeDtypeStruct((BATCH, 1), jnp.float32),
    )(u, v, mu, mv, w1a, w1b, b1.reshape(1, 256), W2, b2.reshape(1, 128),
      Wf, bf.reshape(1, 1))


def kernel(users, items, user_emb, item_emb, W1, b1, W2, b2, Wf, bf):
    users = users.astype(jnp.int32)
    items = items.astype(jnp.int32)
    eye16 = jnp.eye(128, dtype=jnp.bfloat16)
    utab = _pair_table(user_emb.T, eye16)
    itab = _pair_table(item_emb.T, eye16)
    u, v = _sc_gather(users, items, utab, itab)
    mu = (users >= HP).astype(jnp.float32).reshape(BATCH, 1)
    mv = (items >= HP).astype(jnp.float32).reshape(BATCH, 1)
    out = _mlp(u, v, mu, mv, W1, b1, W2, b2, Wf, bf)
    return out.reshape(BATCH)


# speedup vs baseline: 2.1956x; 1.3607x over previous
import functools

import jax
import jax.numpy as jnp
from jax import lax
from jax.experimental import pallas as pl
from jax.experimental.pallas import tpu as pltpu
from jax.experimental.pallas import tpu_sc as plsc

N_FACTORS = 64
N_ROWS = 1000001
BATCH = 16384
TB = 4096
NTB = 123
HP = TB * NTB
NC, NS = 2, 16
NW = NC * NS
BPW = BATCH // NW

_sc_mesh = plsc.VectorSubcoreMesh(core_axis_name="c", subcore_axis_name="s")


@functools.partial(
    pl.kernel,
    mesh=_sc_mesh,
    compiler_params=pltpu.CompilerParams(use_tc_tiling_on_sc=True),
    out_type=(
        jax.ShapeDtypeStruct((BATCH, 128), jnp.float32),
        jax.ShapeDtypeStruct((BATCH, 128), jnp.float32),
    ),
    scratch_types=[
        pltpu.VMEM((BPW,), jnp.int32),
        pltpu.VMEM((BPW,), jnp.int32),
        pltpu.VMEM((BPW, 128), jnp.float32),
        pltpu.SemaphoreType.DMA,
    ],
)
def _sc_gather(users_hbm, items_hbm, utab_hbm, itab_hbm, u_out, v_out,
               uidx_v, iidx_v, rows_v, sem):
    wid = lax.axis_index("s") * NC + lax.axis_index("c")
    base = wid * BPW
    pltpu.sync_copy(users_hbm.at[pl.ds(base, BPW)], uidx_v)
    pltpu.sync_copy(items_hbm.at[pl.ds(base, BPW)], iidx_v)

    def gather_one(tab, idx_v, out):
        def body(g, _):
            vec = idx_v[pl.ds(g * 16, 16)]
            vec = jnp.where(vec >= HP, vec - HP, vec)
            for l in range(16):
                p = vec[l]
                pltpu.make_async_copy(tab.at[p], rows_v.at[g * 16 + l],
                                      sem).start()
            return 0
        lax.fori_loop(0, BPW // 16, body, 0)
        pltpu.make_async_copy(out.at[pl.ds(0, BPW)], rows_v, sem).wait()
        pltpu.sync_copy(rows_v, out.at[pl.ds(base, BPW)])

    gather_one(utab_hbm, uidx_v, u_out)
    gather_one(itab_hbm, iidx_v, v_out)


def _pair_body(a_ref, b_ref, eye_ref, out_ref):
    dn = (((0,), (0,)), ((), ()))
    c16 = jnp.concatenate(
        [a_ref[...], b_ref[...]], axis=0).astype(jnp.bfloat16)
    out_ref[...] = lax.dot_general(
        c16, eye_ref[...], dn, preferred_element_type=jnp.float32)


def _pair_table(tabT, eye16):
    return pl.pallas_call(
        _pair_body,
        grid=(NTB,),
        in_specs=[
            pl.BlockSpec((N_FACTORS, TB), lambda i: (0, i)),
            pl.BlockSpec((N_FACTORS, TB),
                         lambda i: (0, jnp.minimum(i + NTB, NTB * 2 - 2))),
            pl.BlockSpec((128, 128), lambda i: (0, 0)),
        ],
        out_specs=pl.BlockSpec((TB, 128), lambda i: (i, 0)),
        out_shape=jax.ShapeDtypeStruct((HP, 128), jnp.float32),
    )(tabT, tabT, eye16)


BLK = 2048


def _mlp_body(u_ref, v_ref, mu_ref, mv_ref, w1a_ref, w1b_ref, b1_ref,
              w2_ref, b2_ref, wf_ref, bf_ref, out_ref):
    u = jnp.where(mu_ref[...] > 0.5, u_ref[:, N_FACTORS:],
                  u_ref[:, :N_FACTORS])
    v = jnp.where(mv_ref[...] > 0.5, v_ref[:, N_FACTORS:],
                  v_ref[:, :N_FACTORS])
    h = jnp.dot(u, w1a_ref[...], preferred_element_type=jnp.float32)
    h += jnp.dot(v, w1b_ref[...], preferred_element_type=jnp.float32)
    h = jnp.maximum(h + b1_ref[...], 0.0)
    h = jnp.dot(h, w2_ref[...], preferred_element_type=jnp.float32)
    h = jnp.maximum(h + b2_ref[...], 0.0)
    out_ref[...] = jnp.dot(h, wf_ref[...],
                           preferred_element_type=jnp.float32) + bf_ref[...]


def _mlp(u, v, mu, mv, W1, b1, W2, b2, Wf, bf):
    w1a, w1b = W1[:N_FACTORS], W1[N_FACTORS:]
    grid = (BATCH // BLK,)
    fixed = lambda i: (0, 0)
    return pl.pallas_call(
        _mlp_body,
        grid=grid,
        in_specs=[
            pl.BlockSpec((BLK, 128), lambda i: (i, 0)),
            pl.BlockSpec((BLK, 128), lambda i: (i, 0)),
            pl.BlockSpec((BLK, 1), lambda i: (i, 0)),
            pl.BlockSpec((BLK, 1), lambda i: (i, 0)),
            pl.BlockSpec((N_FACTORS, 256), fixed),
            pl.BlockSpec((N_FACTORS, 256), fixed),
            pl.BlockSpec((1, 256), fixed),
            pl.BlockSpec((256, 128), fixed),
            pl.BlockSpec((1, 128), fixed),
            pl.BlockSpec((128, 1), fixed),
            pl.BlockSpec((1, 1), fixed),
        ],
        out_specs=pl.BlockSpec((BLK, 1), lambda i: (i, 0)),
        out_shape=jax.ShapeDtypeStruct((BATCH, 1), jnp.float32),
    )(u, v, mu, mv, w1a, w1b, b1.reshape(1, 256), W2, b2.reshape(1, 128),
      Wf, bf.reshape(1, 1))


def kernel(users, items, user_emb, item_emb, W1, b1, W2, b2, Wf, bf):
    users = users.astype(jnp.int32)
    items = items.astype(jnp.int32)
    eye16 = jnp.eye(128, dtype=jnp.bfloat16)
    utab = _pair_table(user_emb.T, eye16)
    itab = _pair_table(item_emb.T, eye16)
    u, v = _sc_gather(users, items, utab, itab)
    mu = (users >= HP).astype(jnp.float32).reshape(BATCH, 1)
    mv = (items >= HP).astype(jnp.float32).reshape(BATCH, 1)
    out = _mlp(u, v, mu, mv, W1, b1, W2, b2, Wf, bf)
    return out.reshape(BATCH)


# speedup vs baseline: 2.4858x; 1.1322x over previous
import functools

import jax
import jax.numpy as jnp
from jax import lax
from jax.experimental import pallas as pl
from jax.experimental.pallas import tpu as pltpu
from jax.experimental.pallas import tpu_sc as plsc

N_FACTORS = 64
N_ROWS = 1000001
BATCH = 16384
TB = 8192
NTB = 62
HP = TB * NTB
NC, NS = 2, 16
NW = NC * NS
BPW = BATCH // NW

_sc_mesh = plsc.VectorSubcoreMesh(core_axis_name="c", subcore_axis_name="s")


@functools.partial(
    pl.kernel,
    mesh=_sc_mesh,
    compiler_params=pltpu.CompilerParams(use_tc_tiling_on_sc=True),
    out_type=(
        jax.ShapeDtypeStruct((BATCH, 128), jnp.float32),
        jax.ShapeDtypeStruct((BATCH, 128), jnp.float32),
    ),
    scratch_types=[
        pltpu.VMEM((BPW,), jnp.int32),
        pltpu.VMEM((BPW,), jnp.int32),
        pltpu.VMEM((BPW, 128), jnp.float32),
        pltpu.SemaphoreType.DMA,
    ],
)
def _sc_gather(users_hbm, items_hbm, utab_hbm, itab_hbm, u_out, v_out,
               uidx_v, iidx_v, rows_v, sem):
    wid = lax.axis_index("s") * NC + lax.axis_index("c")
    base = wid * BPW
    pltpu.sync_copy(users_hbm.at[pl.ds(base, BPW)], uidx_v)
    pltpu.sync_copy(items_hbm.at[pl.ds(base, BPW)], iidx_v)

    def gather_one(tab, idx_v, out):
        def body(g, _):
            vec = idx_v[pl.ds(g * 16, 16)]
            vec = jnp.where(vec >= HP, vec - HP, vec)
            for l in range(16):
                p = vec[l]
                pltpu.make_async_copy(tab.at[p], rows_v.at[g * 16 + l],
                                      sem).start()
            return 0
        lax.fori_loop(0, BPW // 16, body, 0)
        pltpu.make_async_copy(out.at[pl.ds(0, BPW)], rows_v, sem).wait()
        pltpu.sync_copy(rows_v, out.at[pl.ds(base, BPW)])

    gather_one(utab_hbm, uidx_v, u_out)
    gather_one(itab_hbm, iidx_v, v_out)


def _pair_body(a_ref, b_ref, eye_ref, out_ref):
    dn = (((0,), (0,)), ((), ()))
    c16 = jnp.concatenate(
        [a_ref[...], b_ref[...]], axis=0).astype(jnp.bfloat16)
    out_ref[...] = lax.dot_general(
        c16, eye_ref[...], dn, preferred_element_type=jnp.float32)


def _pair_table(tabT, eye16):
    return pl.pallas_call(
        _pair_body,
        grid=(NTB,),
        in_specs=[
            pl.BlockSpec((N_FACTORS, TB), lambda i: (0, i)),
            pl.BlockSpec((N_FACTORS, TB),
                         lambda i: (0, jnp.minimum(i + NTB, NTB * 2 - 2))),
            pl.BlockSpec((128, 128), lambda i: (0, 0)),
        ],
        out_specs=pl.BlockSpec((TB, 128), lambda i: (i, 0)),
        out_shape=jax.ShapeDtypeStruct((HP, 128), jnp.float32),
    )(tabT, tabT, eye16)


BLK = 2048


def _mlp_body(u_ref, v_ref, mu_ref, mv_ref, w1a_ref, w1b_ref, b1_ref,
              w2_ref, b2_ref, wf_ref, bf_ref, out_ref):
    u = jnp.where(mu_ref[...] > 0.5, u_ref[:, N_FACTORS:],
                  u_ref[:, :N_FACTORS])
    v = jnp.where(mv_ref[...] > 0.5, v_ref[:, N_FACTORS:],
                  v_ref[:, :N_FACTORS])
    h = jnp.dot(u, w1a_ref[...], preferred_element_type=jnp.float32)
    h += jnp.dot(v, w1b_ref[...], preferred_element_type=jnp.float32)
    h = jnp.maximum(h + b1_ref[...], 0.0)
    h = jnp.dot(h, w2_ref[...], preferred_element_type=jnp.float32)
    h = jnp.maximum(h + b2_ref[...], 0.0)
    out_ref[...] = jnp.dot(h, wf_ref[...],
                           preferred_element_type=jnp.float32) + bf_ref[...]


def _mlp(u, v, mu, mv, W1, b1, W2, b2, Wf, bf):
    w1a, w1b = W1[:N_FACTORS], W1[N_FACTORS:]
    grid = (BATCH // BLK,)
    fixed = lambda i: (0, 0)
    return pl.pallas_call(
        _mlp_body,
        grid=grid,
        in_specs=[
            pl.BlockSpec((BLK, 128), lambda i: (i, 0)),
            pl.BlockSpec((BLK, 128), lambda i: (i, 0)),
            pl.BlockSpec((BLK, 1), lambda i: (i, 0)),
            pl.BlockSpec((BLK, 1), lambda i: (i, 0)),
            pl.BlockSpec((N_FACTORS, 256), fixed),
            pl.BlockSpec((N_FACTORS, 256), fixed),
            pl.BlockSpec((1, 256), fixed),
            pl.BlockSpec((256, 128), fixed),
            pl.BlockSpec((1, 128), fixed),
            pl.BlockSpec((128, 1), fixed),
            pl.BlockSpec((1, 1), fixed),
        ],
        out_specs=pl.BlockSpec((BLK, 1), lambda i: (i, 0)),
        out_shape=jax.ShapeDtypeStruct((BATCH, 1), jnp.float32),
    )(u, v, mu, mv, w1a, w1b, b1.reshape(1, 256), W2, b2.reshape(1, 128),
      Wf, bf.reshape(1, 1))


def kernel(users, items, user_emb, item_emb, W1, b1, W2, b2, Wf, bf):
    users = users.astype(jnp.int32)
    items = items.astype(jnp.int32)
    eye16 = jnp.eye(128, dtype=jnp.bfloat16)
    utab = _pair_table(user_emb.T, eye16)
    itab = _pair_table(item_emb.T, eye16)
    u, v = _sc_gather(users, items, utab, itab)
    mu = (users >= HP).astype(jnp.float32).reshape(BATCH, 1)
    mv = (items >= HP).astype(jnp.float32).reshape(BATCH, 1)
    out = _mlp(u, v, mu, mv, W1, b1, W2, b2, Wf, bf)
    return out.reshape(BATCH)
